# Initial kernel scaffold; baseline (speedup 1.0000x reference)
import jax
import jax.numpy as jnp
from jax.experimental import pallas as pl


def kernel(cls_pred, reg_pred, anchors):
    raise NotImplementedError("write your pallas kernel here")


# speedup vs baseline: 2.5656x; 2.5656x over previous
import jax
import jax.numpy as jnp
from jax.experimental import pallas as pl
from jax.experimental.pallas import tpu as pltpu

_NUM_CLASSES = 20
_TOPK = 1000
_PAD = 1024
_CONF_THRESH = 0.05
_NMS_THRESH = 0.6
_STRIDE = 8.0


def _nms_kernel(sc_r, lab_r, lab_c, anc_c, reg_c, anc_r, reg_r, out_ref, iou_s):
    ac = anc_c[:]
    rc = reg_c[:]
    ctr_c = ac + rc[:, :2] * _STRIDE
    wh_c = jnp.exp(rc[:, 2:]) * _STRIDE
    p1_c = ctr_c - 0.5 * wh_c
    p2_c = ctr_c + 0.5 * wh_c
    boxes_c = jnp.concatenate([p1_c, p2_c], axis=1)

    ar = anc_r[:]
    rr = reg_r[:]
    ctr_r = ar + rr[:2, :] * _STRIDE
    wh_r = jnp.exp(rr[2:, :]) * _STRIDE
    p1_r = ctr_r - 0.5 * wh_r
    p2_r = ctr_r + 0.5 * wh_r

    row_ids = jax.lax.broadcasted_iota(jnp.int32, (_PAD, 1), 0)
    real_c = row_ids < _TOPK
    mc = jnp.max(jnp.where(real_c, boxes_c, -1e30))

    off_c = lab_c[:] * (mc + 1.0)
    off_r = lab_r[:] * (mc + 1.0)
    x1c = boxes_c[:, 0:1] + off_c
    y1c = boxes_c[:, 1:2] + off_c
    x2c = boxes_c[:, 2:3] + off_c
    y2c = boxes_c[:, 3:4] + off_c
    x1r = p1_r[0:1, :] + off_r
    y1r = p1_r[1:2, :] + off_r
    x2r = p2_r[0:1, :] + off_r
    y2r = p2_r[1:2, :] + off_r

    area_c = jnp.maximum(x2c - x1c, 0.0) * jnp.maximum(y2c - y1c, 0.0)
    area_r = jnp.maximum(x2r - x1r, 0.0) * jnp.maximum(y2r - y1r, 0.0)
    xx1 = jnp.maximum(x1c, x1r)
    yy1 = jnp.maximum(y1c, y1r)
    xx2 = jnp.minimum(x2c, x2r)
    yy2 = jnp.minimum(y2c, y2r)
    inter = jnp.maximum(xx2 - xx1, 0.0) * jnp.maximum(yy2 - yy1, 0.0)
    union = area_c + area_r - inter
    iou_s[:, :] = inter / jnp.maximum(union, 1e-9)

    scores = sc_r[:]
    col_ids = jax.lax.broadcasted_iota(jnp.int32, (1, _PAD), 1)
    keep0 = jnp.where(scores > _CONF_THRESH, 1.0, 0.0)

    def body(i, keep):
        row = iou_s[pl.ds(i, 1), :]
        ki = jnp.max(jnp.where(col_ids == i, keep, 0.0))
        sup = (row > _NMS_THRESH) & (col_ids > i) & (ki > 0.0)
        return jnp.where(sup, 0.0, keep)

    keep = jax.lax.fori_loop(0, _TOPK, body, keep0)

    bx_r = jnp.concatenate([p1_r, p2_r], axis=0)
    out_ref[0:4, :] = bx_r * keep
    out_ref[4:5, :] = scores * keep
    out_ref[5:6, :] = lab_r[:]


def kernel(cls_pred, reg_pred, anchors):
    scores_flat = jax.nn.sigmoid(cls_pred).reshape(-1)
    topk_scores, topk_idxs = jax.lax.top_k(scores_flat, _TOPK)
    anchor_idxs = topk_idxs // _NUM_CLASSES
    labels = (topk_idxs % _NUM_CLASSES).astype(jnp.float32)
    anc = anchors[anchor_idxs]
    regs = reg_pred[anchor_idxs]

    pad = _PAD - _TOPK
    sc_p = jnp.pad(topk_scores, (0, pad), constant_values=-1.0)
    lab_p = jnp.pad(labels, (0, pad))
    anc_p = jnp.pad(anc, ((0, pad), (0, 0)))
    reg_p = jnp.pad(regs, ((0, pad), (0, 0)))

    out = pl.pallas_call(
        _nms_kernel,
        out_shape=jax.ShapeDtypeStruct((6, _PAD), jnp.float32),
        scratch_shapes=[pltpu.VMEM((_PAD, _PAD), jnp.float32)],
    )(
        sc_p.reshape(1, _PAD),
        lab_p.reshape(1, _PAD),
        lab_p.reshape(_PAD, 1),
        anc_p,
        reg_p,
        anc_p.T,
        reg_p.T,
    )
    return out.T[:_TOPK]


# speedup vs baseline: 3.3542x; 1.3074x over previous
import jax
import jax.numpy as jnp
from jax.experimental import pallas as pl
from jax.experimental.pallas import tpu as pltpu

_NUM_CLASSES = 20
_TOPK = 1000
_PAD = 1024
_CONF_THRESH = 0.05
_NMS_THRESH = 0.6
_STRIDE = 8.0


def _nms_kernel(sc_r, lab_r, lab_c, anc_c, reg_c, anc_r, reg_r, out_ref, iou_s):
    ac = anc_c[:]
    rc = reg_c[:]
    ctr_c = ac + rc[:, :2] * _STRIDE
    wh_c = jnp.exp(rc[:, 2:]) * _STRIDE
    p1_c = ctr_c - 0.5 * wh_c
    p2_c = ctr_c + 0.5 * wh_c
    boxes_c = jnp.concatenate([p1_c, p2_c], axis=1)

    ar = anc_r[:]
    rr = reg_r[:]
    ctr_r = ar + rr[:2, :] * _STRIDE
    wh_r = jnp.exp(rr[2:, :]) * _STRIDE
    p1_r = ctr_r - 0.5 * wh_r
    p2_r = ctr_r + 0.5 * wh_r

    row_ids = jax.lax.broadcasted_iota(jnp.int32, (_PAD, 1), 0)
    real_c = row_ids < _TOPK
    mc = jnp.max(jnp.where(real_c, boxes_c, -1e30))

    off_c = lab_c[:] * (mc + 1.0)
    off_r = lab_r[:] * (mc + 1.0)
    x1c = boxes_c[:, 0:1] + off_c
    y1c = boxes_c[:, 1:2] + off_c
    x2c = boxes_c[:, 2:3] + off_c
    y2c = boxes_c[:, 3:4] + off_c
    x1r = p1_r[0:1, :] + off_r
    y1r = p1_r[1:2, :] + off_r
    x2r = p2_r[0:1, :] + off_r
    y2r = p2_r[1:2, :] + off_r

    area_c = jnp.maximum(x2c - x1c, 0.0) * jnp.maximum(y2c - y1c, 0.0)
    area_r = jnp.maximum(x2r - x1r, 0.0) * jnp.maximum(y2r - y1r, 0.0)
    xx1 = jnp.maximum(x1c, x1r)
    yy1 = jnp.maximum(y1c, y1r)
    xx2 = jnp.minimum(x2c, x2r)
    yy2 = jnp.minimum(y2c, y2r)
    inter = jnp.maximum(xx2 - xx1, 0.0) * jnp.maximum(yy2 - yy1, 0.0)
    union = area_c + area_r - inter
    iou_s[:, :] = inter / jnp.maximum(union, 1e-9)

    scores = sc_r[:]
    col_ids = jax.lax.broadcasted_iota(jnp.int32, (1, _PAD), 1)
    keep0 = jnp.where(scores > _CONF_THRESH, 1.0, 0.0)

    def body(i, keep):
        row = iou_s[pl.ds(i, 1), :]
        ki = jnp.max(jnp.where(col_ids == i, keep, 0.0))
        sup = (row > _NMS_THRESH) & (col_ids > i) & (ki > 0.0)
        return jnp.where(sup, 0.0, keep)

    keep = jax.lax.fori_loop(0, _TOPK, body, keep0)

    bx_r = jnp.concatenate([p1_r, p2_r], axis=0)
    out_ref[0:4, :] = bx_r * keep
    out_ref[4:5, :] = scores * keep
    out_ref[5:6, :] = lab_r[:]


def kernel(cls_pred, reg_pred, anchors):
    scores_flat = jax.nn.sigmoid(cls_pred).reshape(-1)
    n_chunks = 8
    chunk = scores_flat.shape[0] // n_chunks
    tv, ti = jax.lax.top_k(scores_flat.reshape(n_chunks, chunk), _TOPK)
    gidx = (ti + (jnp.arange(n_chunks, dtype=ti.dtype) * chunk)[:, None]).reshape(-1)
    topk_scores, sel = jax.lax.top_k(tv.reshape(-1), _TOPK)
    topk_idxs = gidx[sel]
    anchor_idxs = topk_idxs // _NUM_CLASSES
    labels = (topk_idxs % _NUM_CLASSES).astype(jnp.float32)
    anc = anchors[anchor_idxs]
    regs = reg_pred[anchor_idxs]

    pad = _PAD - _TOPK
    sc_p = jnp.pad(topk_scores, (0, pad), constant_values=-1.0)
    lab_p = jnp.pad(labels, (0, pad))
    anc_p = jnp.pad(anc, ((0, pad), (0, 0)))
    reg_p = jnp.pad(regs, ((0, pad), (0, 0)))

    out = pl.pallas_call(
        _nms_kernel,
        out_shape=jax.ShapeDtypeStruct((6, _PAD), jnp.float32),
        scratch_shapes=[pltpu.VMEM((_PAD, _PAD), jnp.float32)],
    )(
        sc_p.reshape(1, _PAD),
        lab_p.reshape(1, _PAD),
        lab_p.reshape(_PAD, 1),
        anc_p,
        reg_p,
        anc_p.T,
        reg_p.T,
    )
    return out.T[:_TOPK]
